# Initial kernel scaffold; baseline (speedup 1.0000x reference)
import jax
import jax.numpy as jnp
from jax.experimental import pallas as pl


def kernel(u_mix, u_exp, eps_gauss, prior):
    raise NotImplementedError("write your pallas kernel here")


# speedup vs baseline: 1.0151x; 1.0151x over previous
import jax
import jax.numpy as jnp
from jax.experimental import pallas as pl
from jax.experimental.pallas import tpu as pltpu

_RATE = 1.0
_MEAN = 0.5
_STD = 0.2
_UPPER = 10.0


def _mix_body(prior_ref, u_mix_ref, u_exp_ref, eps_ref, out_ref):
    p0 = prior_ref[0] / (prior_ref[0] + prior_ref[1])
    gauss = _MEAN + _STD * eps_ref[...]
    exp_s = -jnp.log1p(-u_exp_ref[...]) / _RATE
    out = jnp.where(u_mix_ref[...] >= p0, gauss, exp_s)
    out_ref[...] = jnp.clip(out, 0.0, _UPPER)


def kernel(u_mix, u_exp, eps_gauss, prior):
    R, C = u_mix.shape
    BC = 4096
    grid = (C // BC,)
    bspec = pl.BlockSpec((R, BC), lambda i: (0, i))
    return pl.pallas_call(
        _mix_body,
        grid=grid,
        in_specs=[
            pl.BlockSpec(memory_space=pltpu.SMEM),
            bspec,
            bspec,
            bspec,
        ],
        out_specs=bspec,
        out_shape=jax.ShapeDtypeStruct((R, C), jnp.float32),
    )(prior, u_mix, u_exp, eps_gauss)
